# baseline (speedup 1.0000x reference)
import functools

import jax
import jax.numpy as jnp
from jax import lax
from jax.experimental import pallas as pl
from jax.experimental.pallas import tpu as pltpu

N, B, OS, CS, H = 32768, 16, 64, 128, 64
N2 = N // 2
NP = 2048
NB = N2 // NP
NP2 = 8192
NB2 = N2 // NP2
NEG = -1e30

_NT = (((1,), (1,)), ((), ()))


def _init_body(ctx_ref, Wq_ref, bq_ref, W3k_ref, b3k_ref, wlog_ref, blog_ref):
    f32 = jnp.float32
    q_all = jnp.dot(ctx_ref[...], Wq_ref[...], preferred_element_type=f32) + bq_ref[...]
    wlog_ref[...] = (lax.dot_general(q_all, W3k_ref[...], _NT,
                                     preferred_element_type=f32) * 0.125
                     ).astype(jnp.bfloat16)
    blog_ref[...] = jnp.sum(q_all * b3k_ref[...], axis=1,
                            keepdims=True) * 0.125


def _pass1_body(obj_ref, seg_ref, W1_ref, b1_ref, W2_ref, b2_ref,
                W3v_ref, b3v_ref, wlog_ref, blog_ref,
                logits_ref, m_ref, d_ref, emb_ref,
                acc_ref, ms_ref, ds_ref):
    i = pl.program_id(0)
    f32 = jnp.float32
    bf16 = jnp.bfloat16

    @pl.when(i == 0)
    def _init():
        ms_ref[...] = jnp.full((B, 1), NEG, f32)
        ds_ref[...] = jnp.zeros((B, 1), f32)
        acc_ref[...] = jnp.zeros((B, H), f32)

    obj = obj_ref[...]
    h = jnp.dot(obj, W1_ref[...], preferred_element_type=f32) + b1_ref[...]
    h = jnp.maximum(h, 0.0).astype(bf16)
    h = jnp.dot(h, W2_ref[...], preferred_element_type=f32) + b2_ref[...]
    h = jnp.maximum(h, 0.0).astype(bf16)
    value = (jnp.dot(h, W3v_ref[...], preferred_element_type=f32)
             + b3v_ref[...]).astype(bf16)

    lt = lax.dot_general(wlog_ref[...], h, _NT,
                         preferred_element_type=f32) + blog_ref[...]

    se = seg_ref[0:1, :]
    so = seg_ref[1:2, :]
    iota = lax.broadcasted_iota(jnp.int32, (B, NP), 0)
    maske = se == iota
    masko = so == iota
    mask = jnp.concatenate([maske, masko], axis=0)
    masked = jnp.where(mask, lt, -jnp.inf)

    le = jnp.sum(jnp.where(maske, lt[:B, :], 0.0), axis=0, keepdims=True)
    lo = jnp.sum(jnp.where(masko, lt[B:, :], 0.0), axis=0, keepdims=True)
    logits_ref[...] = jnp.concatenate([le, lo], axis=0)

    bm32 = jnp.max(masked, axis=1, keepdims=True)
    bm = jnp.maximum(bm32[:B, :], bm32[B:, :])
    m_old = ms_ref[...]
    m_new = jnp.maximum(m_old, bm)
    alpha = jnp.exp(m_old - m_new)
    ms_ref[...] = m_new
    mfull = jnp.concatenate([m_new, m_new], axis=0)
    e = jnp.exp(masked - mfull)
    es = jnp.sum(e, axis=1, keepdims=True)
    ds_ref[...] = ds_ref[...] * alpha + es[:B, :] + es[B:, :]
    accf = jnp.dot(e.astype(bf16), value, preferred_element_type=f32)
    acc_ref[...] = acc_ref[...] * alpha + accf[:B, :H] + accf[B:, H:]

    d = ds_ref[...]
    m_ref[...] = ms_ref[...]
    d_ref[...] = d
    emb_ref[...] = jnp.where(d > 0.0, acc_ref[...] / d, 0.0)


def _pass2_body(logits_ref, seg_ref, m_ref, d_ref, w_ref):
    se = seg_ref[0:1, :]
    so = seg_ref[1:2, :]
    iota = lax.broadcasted_iota(jnp.int32, (B, NP2), 0)
    maske = se == iota
    masko = so == iota
    mb = jnp.broadcast_to(m_ref[...], (B, NP2))
    rb = jnp.broadcast_to(1.0 / d_ref[...], (B, NP2))
    mge = jnp.sum(jnp.where(maske, mb, 0.0), axis=0, keepdims=True)
    mgo = jnp.sum(jnp.where(masko, mb, 0.0), axis=0, keepdims=True)
    rge = jnp.sum(jnp.where(maske, rb, 0.0), axis=0, keepdims=True)
    rgo = jnp.sum(jnp.where(masko, rb, 0.0), axis=0, keepdims=True)
    l = logits_ref[...]
    we = jnp.exp(l[0:1, :] - mge) * rge
    wo = jnp.exp(l[1:2, :] - mgo) * rgo
    w_ref[...] = jnp.concatenate([we, wo], axis=0)


def _blockdiag2(a):
    z = jnp.zeros_like(a)
    return jnp.concatenate([jnp.concatenate([a, z], axis=1),
                            jnp.concatenate([z, a], axis=1)], axis=0)


@jax.jit
def kernel(objects, context, segment_ids, W1, b1, W2, b2, W3, b3, Wq, bq):
    bf16 = jnp.bfloat16
    obj2 = objects.reshape(N2, 2 * OS).astype(bf16)
    segp = segment_ids.reshape(N2, 2).T
    W3k, W3v = W3[:, :H], W3[:, H:]
    b3k = b3[:H].reshape(1, H)
    W1d = _blockdiag2(W1).astype(bf16)
    W2d = _blockdiag2(W2).astype(bf16)
    W3vd = _blockdiag2(W3v).astype(bf16)
    b1d = jnp.concatenate([b1, b1]).reshape(1, 2 * H)
    b2d = jnp.concatenate([b2, b2]).reshape(1, 2 * H)
    b3vd = jnp.concatenate([b3[H:], b3[H:]]).reshape(1, 2 * H)
    bqr = bq.reshape(1, H)

    full = lambda s: pl.BlockSpec(s, lambda i: tuple(0 for _ in s))

    wlog, blog = pl.pallas_call(
        _init_body,
        grid=(1,),
        in_specs=[full((B, CS)), full((CS, H)), full((1, H)),
                  full((H, H)), full((1, H))],
        out_specs=[full((B, H)), full((B, 1))],
        out_shape=[
            jax.ShapeDtypeStruct((B, H), jnp.bfloat16),
            jax.ShapeDtypeStruct((B, 1), jnp.float32),
        ],
    )(context, Wq, bqr, W3k, b3k)

    wlogd = _blockdiag2(wlog)
    blogd = jnp.concatenate([blog, blog], axis=0)

    logits2, m, d, emb = pl.pallas_call(
        _pass1_body,
        grid=(NB,),
        in_specs=[
            pl.BlockSpec((NP, 2 * OS), lambda i: (i, 0)),
            pl.BlockSpec((2, NP), lambda i: (0, i)),
            full((2 * OS, 2 * H)), full((1, 2 * H)),
            full((2 * H, 2 * H)), full((1, 2 * H)),
            full((2 * H, 2 * H)), full((1, 2 * H)),
            full((2 * B, 2 * H)), full((2 * B, 1)),
        ],
        out_specs=[
            pl.BlockSpec((2, NP), lambda i: (0, i)),
            full((B, 1)), full((B, 1)), full((B, H)),
        ],
        out_shape=[
            jax.ShapeDtypeStruct((2, N2), jnp.float32),
            jax.ShapeDtypeStruct((B, 1), jnp.float32),
            jax.ShapeDtypeStruct((B, 1), jnp.float32),
            jax.ShapeDtypeStruct((B, H), jnp.float32),
        ],
        scratch_shapes=[
            pltpu.VMEM((B, H), jnp.float32),
            pltpu.VMEM((B, 1), jnp.float32),
            pltpu.VMEM((B, 1), jnp.float32),
        ],
        compiler_params=pltpu.CompilerParams(
            dimension_semantics=("arbitrary",),
        ),
    )(obj2, segp, W1d, b1d, W2d, b2d, W3vd, b3vd, wlogd, blogd)

    w2 = pl.pallas_call(
        _pass2_body,
        grid=(NB2,),
        in_specs=[
            pl.BlockSpec((2, NP2), lambda i: (0, i)),
            pl.BlockSpec((2, NP2), lambda i: (0, i)),
            full((B, 1)), full((B, 1)),
        ],
        out_specs=pl.BlockSpec((2, NP2), lambda i: (0, i)),
        out_shape=jax.ShapeDtypeStruct((2, N2), jnp.float32),
        compiler_params=pltpu.CompilerParams(
            dimension_semantics=("arbitrary",),
        ),
    )(logits2, segp, m, d)

    return emb, w2.T.reshape(N)


# speedup vs baseline: 2.1519x; 2.1519x over previous
import functools

import jax
import jax.numpy as jnp
from jax import lax
from jax.experimental import pallas as pl
from jax.experimental.pallas import tpu as pltpu

N, B, OS, CS, H = 32768, 16, 64, 128, 64
BN = 4096
NB = N // BN
BN2 = 16384
NB2 = N // BN2
NEG = -1e30

_NT = (((1,), (1,)), ((), ()))


def _init_body(ctx_ref, Wq_ref, bq_ref, W3k_ref, b3k_ref, wlog_ref, blog_ref):
    f32 = jnp.float32
    q_all = jnp.dot(ctx_ref[...], Wq_ref[...], preferred_element_type=f32) + bq_ref[...]
    wlog_ref[...] = (lax.dot_general(q_all, W3k_ref[...], _NT,
                                     preferred_element_type=f32) * 0.125
                     ).astype(jnp.bfloat16)
    blog_ref[...] = jnp.sum(q_all * b3k_ref[...], axis=1,
                            keepdims=True) * 0.125


def _pass1_body(obj_ref, seg_ref, W1_ref, b1_ref, W2_ref, b2_ref,
                W3v_ref, b3v_ref, wlog_ref, blog_ref,
                logits_ref, m_ref, d_ref, emb_ref,
                acc_ref, ms_ref, ds_ref):
    i = pl.program_id(0)
    f32 = jnp.float32
    bf16 = jnp.bfloat16

    @pl.when(i == 0)
    def _init():
        ms_ref[...] = jnp.full((B, 1), NEG, f32)
        ds_ref[...] = jnp.zeros((B, 1), f32)
        acc_ref[...] = jnp.zeros((B, H), f32)

    obj = obj_ref[...]
    h = jnp.dot(obj, W1_ref[...], preferred_element_type=f32) + b1_ref[...]
    h = jnp.maximum(h, 0.0).astype(bf16)
    h = jnp.dot(h, W2_ref[...], preferred_element_type=f32) + b2_ref[...]
    h = jnp.maximum(h, 0.0).astype(bf16)
    value = (jnp.dot(h, W3v_ref[...], preferred_element_type=f32)
             + b3v_ref[...]).astype(bf16)

    logits_t = lax.dot_general(wlog_ref[...], h, _NT,
                               preferred_element_type=f32) + blog_ref[...]

    seg = seg_ref[...]
    row = lax.broadcasted_iota(jnp.int32, (B, BN), 0)
    mask = seg == row
    masked = jnp.where(mask, logits_t, -jnp.inf)

    logits_ref[...] = jnp.sum(jnp.where(mask, logits_t, 0.0), axis=0, keepdims=True)

    bm = jnp.max(masked, axis=1, keepdims=True)
    m_old = ms_ref[...]
    m_new = jnp.maximum(m_old, bm)
    alpha = jnp.exp(m_old - m_new)
    ms_ref[...] = m_new
    e = jnp.exp(masked - m_new)
    ds_ref[...] = ds_ref[...] * alpha + jnp.sum(e, axis=1, keepdims=True)
    acc_ref[...] = acc_ref[...] * alpha + jnp.dot(
        e.astype(bf16), value, preferred_element_type=f32)

    d = ds_ref[...]
    m_ref[...] = ms_ref[...]
    d_ref[...] = d
    emb_ref[...] = jnp.where(d > 0.0, acc_ref[...] / d, 0.0)


def _pass2_body(logits_ref, seg_ref, m_ref, d_ref, w_ref):
    seg = seg_ref[...]
    row = lax.broadcasted_iota(jnp.int32, (B, BN2), 0)
    mask = seg == row
    mb = jnp.broadcast_to(m_ref[...], (B, BN2))
    rb = jnp.broadcast_to(1.0 / d_ref[...], (B, BN2))
    mg = jnp.sum(jnp.where(mask, mb, 0.0), axis=0, keepdims=True)
    rg = jnp.sum(jnp.where(mask, rb, 0.0), axis=0, keepdims=True)
    w_ref[...] = jnp.exp(logits_ref[...] - mg) * rg


@jax.jit
def kernel(objects, context, segment_ids, W1, b1, W2, b2, W3, b3, Wq, bq):
    seg2d = segment_ids.reshape(1, N)
    bf16 = jnp.bfloat16
    obj_b = objects.astype(bf16)
    W3k, W3v = W3[:, :H], W3[:, H:].astype(bf16)
    b3k, b3v = b3[:H].reshape(1, H), b3[H:].reshape(1, H)
    b1r, b2r, bqr = b1.reshape(1, H), b2.reshape(1, H), bq.reshape(1, H)
    W1b, W2b = W1.astype(bf16), W2.astype(bf16)

    full = lambda s: pl.BlockSpec(s, lambda i: tuple(0 for _ in s))

    wlog, blog = pl.pallas_call(
        _init_body,
        grid=(1,),
        in_specs=[full((B, CS)), full((CS, H)), full((1, H)),
                  full((H, H)), full((1, H))],
        out_specs=[full((B, H)), full((B, 1))],
        out_shape=[
            jax.ShapeDtypeStruct((B, H), jnp.bfloat16),
            jax.ShapeDtypeStruct((B, 1), jnp.float32),
        ],
    )(context, Wq, bqr, W3k, b3k)

    logits, m, d, emb = pl.pallas_call(
        _pass1_body,
        grid=(NB,),
        in_specs=[
            pl.BlockSpec((BN, OS), lambda i: (i, 0)),
            pl.BlockSpec((1, BN), lambda i: (0, i)),
            full((OS, H)), full((1, H)),
            full((H, H)), full((1, H)),
            full((H, H)), full((1, H)),
            full((B, H)), full((B, 1)),
        ],
        out_specs=[
            pl.BlockSpec((1, BN), lambda i: (0, i)),
            full((B, 1)), full((B, 1)), full((B, H)),
        ],
        out_shape=[
            jax.ShapeDtypeStruct((1, N), jnp.float32),
            jax.ShapeDtypeStruct((B, 1), jnp.float32),
            jax.ShapeDtypeStruct((B, 1), jnp.float32),
            jax.ShapeDtypeStruct((B, H), jnp.float32),
        ],
        scratch_shapes=[
            pltpu.VMEM((B, H), jnp.float32),
            pltpu.VMEM((B, 1), jnp.float32),
            pltpu.VMEM((B, 1), jnp.float32),
        ],
        compiler_params=pltpu.CompilerParams(
            dimension_semantics=("arbitrary",),
        ),
    )(obj_b, seg2d, W1b, b1r, W2b, b2r, W3v, b3v, wlog, blog)

    weights = pl.pallas_call(
        _pass2_body,
        grid=(NB2,),
        in_specs=[
            pl.BlockSpec((1, BN2), lambda i: (0, i)),
            pl.BlockSpec((1, BN2), lambda i: (0, i)),
            full((B, 1)), full((B, 1)),
        ],
        out_specs=pl.BlockSpec((1, BN2), lambda i: (0, i)),
        out_shape=jax.ShapeDtypeStruct((1, N), jnp.float32),
        compiler_params=pltpu.CompilerParams(
            dimension_semantics=("arbitrary",),
        ),
    )(logits, seg2d, m, d)

    return emb, weights.reshape(N)
